# baseline (speedup 1.0000x reference)
import functools

import jax
import jax.numpy as jnp
from jax import lax
from jax.experimental import pallas as pl
from jax.experimental.pallas import tpu as pltpu
from jax.experimental.pallas import tpu_sc as plsc

VOCAB = 1000000
EMBED_DIM = 64
BATCH = 16384
HIST = 50
HIST_PAD = 56

NUM_CORES = 2
NUM_SUBCORES = 16
NW = NUM_CORES * NUM_SUBCORES
W_BATCH = BATCH // NW
NB = 8
NCH = W_BATCH // NB
NBUF = 2
NG = NCH // NBUF
assert NCH % NBUF == 0


@functools.partial(
    pl.kernel,
    out_type=jax.ShapeDtypeStruct((BATCH, HIST, 128), jnp.float32),
    mesh=plsc.VectorSubcoreMesh(core_axis_name="c", subcore_axis_name="s"),
    scratch_types=[
        [pltpu.VMEM((NB * HIST_PAD,), jnp.int32) for _ in range(NBUF)],
        [pltpu.VMEM((NB, HIST_PAD, 128), jnp.float32)
         for _ in range(NBUF)],
        [pltpu.SemaphoreType.DMA for _ in range(NBUF)],
        [pltpu.SemaphoreType.DMA for _ in range(NBUF)],
    ],
    compiler_params=pltpu.CompilerParams(use_tc_tiling_on_sc=True),
)
def _gather_kernel(table_hbm, idx_hbm, out_hbm, idx_bufs, row_bufs, gsems, ssems):
    wid = lax.axis_index("s") * NUM_CORES + lax.axis_index("c")
    b_base = wid * W_BATCH

    if True:
        def start_chunk(slot, batch0):
            off = batch0 * HIST_PAD
            pltpu.sync_copy(idx_hbm.at[pl.ds(off, NB * HIST_PAD)], idx_bufs[slot])
            for bi in range(NB):
                pltpu.async_copy(
                    table_hbm.at[idx_bufs[slot].at[pl.ds(bi * HIST_PAD, HIST_PAD)]],
                    row_bufs[slot].at[bi],
                    gsems[slot])

        def wait_gathers(slot, batch0):
            for bi in range(NB):
                pltpu.make_async_copy(
                    table_hbm.at[idx_bufs[slot].at[pl.ds(bi * HIST_PAD, HIST_PAD)]],
                    row_bufs[slot].at[bi],
                    gsems[slot]).wait()

        def start_stores(slot, batch0):
            for bi in range(NB):
                pltpu.async_copy(
                    row_bufs[slot].at[bi, pl.ds(0, HIST)],
                    out_hbm.at[batch0 + bi],
                    ssems[slot])

        def wait_stores(slot, batch0):
            for bi in range(NB):
                pltpu.make_async_copy(
                    row_bufs[slot].at[bi, pl.ds(0, HIST)],
                    out_hbm.at[batch0 + bi],
                    ssems[slot]).wait()

        for b in range(NBUF):
            start_chunk(b, b_base + b * NB)

        def group(gi, carry):
            for b in range(NBUF):
                batch0 = b_base + (gi * NBUF + b) * NB
                wait_gathers(b, batch0)
                start_stores(b, batch0)

                @pl.when(gi < NG - 1)
                def _():
                    nxt = batch0 + NBUF * NB
                    off = nxt * HIST_PAD
                    pltpu.sync_copy(
                        idx_hbm.at[pl.ds(off, NB * HIST_PAD)], idx_bufs[b])
                    wait_stores(b, batch0)
                    for bi in range(NB):
                        pltpu.async_copy(
                            table_hbm.at[
                                idx_bufs[b].at[pl.ds(bi * HIST_PAD, HIST_PAD)]],
                            row_bufs[b].at[bi],
                            gsems[b])

            return carry

        lax.fori_loop(0, NG, group, 0)

        for b in range(NBUF):
            batch0 = b_base + ((NG - 1) * NBUF + b) * NB
            wait_stores(b, batch0)


def kernel(input_ids, embedding):
    table_p = jnp.pad(embedding, ((0, 0), (0, 128 - EMBED_DIM)))
    idx = input_ids.astype(jnp.int32)
    idx_pad = jnp.pad(idx, ((0, 0), (0, HIST_PAD - HIST))).reshape(-1)
    out = _gather_kernel(table_p, idx_pad)
    return out[:, :, :EMBED_DIM]


# speedup vs baseline: 4.4761x; 4.4761x over previous
import functools

import jax
import jax.numpy as jnp
from jax import lax
from jax.experimental import pallas as pl
from jax.experimental.pallas import tpu as pltpu
from jax.experimental.pallas import tpu_sc as plsc

VOCAB = 1000000
EMBED_DIM = 64
BATCH = 16384
HIST = 50

TOTAL = BATCH * HIST
NUM_CORES = 2
NUM_SUBCORES = 16
NW = NUM_CORES * NUM_SUBCORES
ROWS_PER_W = TOTAL // NW
CHUNK = 400
NCHUNK = ROWS_PER_W // CHUNK
NBUF = 4
NGROUP = NCHUNK // NBUF
assert NCHUNK % NBUF == 0


@functools.partial(
    pl.kernel,
    out_type=jax.ShapeDtypeStruct((TOTAL, EMBED_DIM), jnp.float32),
    mesh=plsc.VectorSubcoreMesh(core_axis_name="c", subcore_axis_name="s"),
    scratch_types=[
        pltpu.VMEM((ROWS_PER_W,), jnp.int32),
        [pltpu.VMEM((CHUNK, EMBED_DIM), jnp.float32) for _ in range(NBUF)],
        [pltpu.SemaphoreType.DMA for _ in range(NBUF)],
        [pltpu.SemaphoreType.DMA for _ in range(NBUF)],
    ],
    compiler_params=pltpu.CompilerParams(use_tc_tiling_on_sc=False),
)
def _gather_kernel(table_hbm, idx_hbm, out_hbm, idx_all, row_bufs, gsems, ssems):
    wid = lax.axis_index("s") * NUM_CORES + lax.axis_index("c")
    base = wid * ROWS_PER_W

    pltpu.sync_copy(idx_hbm.at[pl.ds(base, ROWS_PER_W)], idx_all)

    def idx_chunk(ci):
        return idx_all.at[pl.ds(ci * CHUNK, CHUNK)]

    for b in range(NBUF):
        pltpu.async_copy(table_hbm.at[idx_chunk(b)], row_bufs[b], gsems[b])

    def group(gi, carry):
        for b in range(NBUF):
            ci = gi * NBUF + b
            off = base + ci * CHUNK
            pltpu.make_async_copy(
                table_hbm.at[idx_chunk(ci)], row_bufs[b], gsems[b]).wait()
            pltpu.async_copy(row_bufs[b], out_hbm.at[pl.ds(off, CHUNK)], ssems[b])

            @pl.when(gi < NGROUP - 1)
            def _():
                pltpu.make_async_copy(
                    row_bufs[b], out_hbm.at[pl.ds(off, CHUNK)], ssems[b]).wait()
                pltpu.async_copy(
                    table_hbm.at[idx_chunk(ci + NBUF)], row_bufs[b], gsems[b])

        return carry

    lax.fori_loop(0, NGROUP, group, 0)

    for b in range(NBUF):
        off = base + ((NGROUP - 1) * NBUF + b) * CHUNK
        pltpu.make_async_copy(
            row_bufs[b], out_hbm.at[pl.ds(off, CHUNK)], ssems[b]).wait()


def kernel(input_ids, embedding):
    table_p = jnp.pad(embedding, ((0, 0), (0, 64))).reshape(2 * VOCAB, EMBED_DIM)
    flat_idx = (input_ids.reshape(-1) * 2).astype(jnp.int32)
    out = _gather_kernel(table_p, flat_idx)
    return out.reshape(BATCH, HIST, EMBED_DIM)
